# Initial kernel scaffold; baseline (speedup 1.0000x reference)
import jax
import jax.numpy as jnp
from jax.experimental import pallas as pl


def kernel(x, token_table, pos_table):
    raise NotImplementedError("write your pallas kernel here")


# speedup vs baseline: 4.6109x; 4.6109x over previous
import functools

import jax
import jax.numpy as jnp
from jax import lax
from jax.experimental import pallas as pl
from jax.experimental.pallas import tpu as pltpu
from jax.experimental.pallas import tpu_sc as plsc


@functools.lru_cache(maxsize=None)
def _build(total_rows, S, D):
    mesh = plsc.VectorSubcoreMesh(core_axis_name="c", subcore_axis_name="s")
    n_workers = mesh.num_cores * mesh.num_subcores
    rows_per_worker = total_rows // n_workers
    seqs_per_worker = rows_per_worker // S
    assert rows_per_worker * n_workers == total_rows
    assert seqs_per_worker * S == rows_per_worker

    @functools.partial(
        pl.kernel,
        out_type=jax.ShapeDtypeStruct((total_rows, D), jnp.float32),
        mesh=mesh,
        scratch_types=[
            pltpu.VMEM((rows_per_worker,), jnp.int32),
            pltpu.VMEM((S, D), jnp.float32),
            pltpu.VMEM((S, D), jnp.float32),
            pltpu.SemaphoreType.DMA,
        ],
    )
    def emb_kernel(x_hbm, tok_hbm, pos_hbm, out_hbm, idx_v, pos_v, buf, sem):
        wid = lax.axis_index("s") * mesh.num_cores + lax.axis_index("c")
        base = wid * rows_per_worker
        pltpu.sync_copy(x_hbm.at[pl.ds(base, rows_per_worker)], idx_v)
        pltpu.sync_copy(pos_hbm.at[pl.ds(0, S)], pos_v)

        @pl.loop(0, seqs_per_worker)
        def seq_loop(s):
            row0 = s * S
            pltpu.async_copy(
                tok_hbm.at[idx_v.at[pl.ds(row0, S)]], buf, sem
            ).wait()

            @pl.loop(0, S)
            def row_loop(r):
                for c in range(D // 16):
                    sl = pl.ds(c * 16, 16)
                    buf[r, sl] = buf[r, sl] + pos_v[r, sl]

            pltpu.sync_copy(buf, out_hbm.at[pl.ds(base + row0, S)])

    return emb_kernel


def kernel(x, token_table, pos_table):
    B, S = x.shape
    D = token_table.shape[1]
    xf = x.reshape(B * S).astype(jnp.int32)
    out = _build(B * S, S, D)(xf, token_table, pos_table)
    return out.reshape(B, S, D)
